# baseline (speedup 1.0000x reference)
import functools

import jax
import jax.numpy as jnp
from jax import lax
from jax.experimental import pallas as pl
from jax.experimental.pallas import tpu as pltpu
from jax.experimental.pallas import tpu_sc as plsc

N_NODES = 10000
N_EDGES = 320000
D_FEAT = 128
D_EDGE = 16
U_DIM = 16
B_GRAPHS = 64
K = 64

NC = 2
NS = 16
NW = NC * NS
E_PER_W = N_EDGES // NW
E_CHUNK = 128
E_ROWS_FULL = 78
E_TAIL = E_PER_W - E_ROWS_FULL * E_CHUNK
E_NCH = E_ROWS_FULL + 1
E_BLOCK_CH = 2
N_CHUNKS_FULL = N_NODES // 128
N_TAIL = N_NODES - N_CHUNKS_FULL * 128
DUMMY = B_GRAPHS


N_EACC = 4


def _sc_body(x_hbm, colidx_hbm, ea_hbm, batch_hbm, pe_hbm, pn_hbm,
             col_v, batch_v, seg_v, rows0_v, rows1_v, xrows_v, nseg_v, ze_v,
             eacc0, eacc1, eacc2, eacc3, nacc,
             sem_misc, sem_in0, sem_in1, sem_add0, sem_add1, sem_add2,
             sem_add3, sem_n):
    c = lax.axis_index("c")
    s = lax.axis_index("s")
    wid = s * NC + c
    ebase = wid * E_PER_W
    rows = (rows0_v, rows1_v)
    eacc = (eacc0, eacc1, eacc2, eacc3)
    sem_in = (sem_in0, sem_in1)
    sem_add = (sem_add0, sem_add1, sem_add2, sem_add3)
    BLK = E_BLOCK_CH * E_CHUNK
    NBLK = 40
    nvalid_tail = E_PER_W - (NBLK - 1) * BLK

    def start_load(blk):
        buf = rows[blk % 2]
        if blk < NBLK - 1:
            return pltpu.async_copy(
                ea_hbm.at[pl.ds(ebase + blk * BLK, BLK), :], buf, sem_in[blk % 2])
        return pltpu.async_copy(
            ea_hbm.at[pl.ds(ebase + (NBLK - 1) * BLK, nvalid_tail), :],
            buf.at[pl.ds(0, nvalid_tail), :], sem_in[blk % 2])

    d_batch = pltpu.async_copy(batch_hbm, batch_v, sem_misc)
    d_col = pltpu.async_copy(
        colidx_hbm.at[pl.ds(ebase, E_PER_W)], col_v, sem_misc)
    d_in0 = start_load(0)
    d_in1 = start_load(1)
    d_nseg0 = pltpu.async_copy(
        batch_hbm.at[pl.ds(wid * 128, 128)], nseg_v.at[0], sem_n)
    d_nseg1 = pltpu.async_copy(
        batch_hbm.at[pl.ds((wid + NW) * 128, 128)], nseg_v.at[1], sem_n)

    @pl.when(s == 0)
    def _zero():
        def zrow(r, carry):
            for k in range(D_FEAT // 16):
                xrows_v[r, pl.ds(k * 16, 16)] = jnp.zeros((16,), jnp.float32)
            ze_v[r, pl.ds(0, 16)] = jnp.zeros((16,), jnp.float32)
            return carry
        lax.fori_loop(0, B_GRAPHS + 1, zrow, 0)
        pltpu.sync_copy(xrows_v.at[pl.ds(0, B_GRAPHS + 1), :], nacc)
        for a in range(N_EACC):
            pltpu.sync_copy(ze_v, eacc[a])

    plsc.subcore_barrier()

    d_batch.wait()
    d_col.wait()

    def seg_row(r, carry):
        for k in range(8):
            col16 = col_v[pl.ds(r * 128 + k * 16, 16)]
            seg_v[r, pl.ds(k * 16, 16)] = plsc.load_gather(batch_v, [col16])
        return carry
    lax.fori_loop(0, E_ROWS_FULL, seg_row, 0)
    col16 = col_v[pl.ds(E_ROWS_FULL * 128, 16)]
    seg_v[E_ROWS_FULL, pl.ds(0, 16)] = plsc.load_gather(batch_v, [col16])
    for k in range(1, 8):
        seg_v[E_ROWS_FULL, pl.ds(k * 16, 16)] = jnp.full((16,), DUMMY, jnp.int32)

    in_desc = [d_in0, d_in1] + [None] * (NBLK - 2)
    add_desc = [None] * N_EACC
    for blk in range(NBLK):
        cur = blk % 2
        in_desc[blk].wait()
        nch = E_BLOCK_CH if blk < NBLK - 1 else 1
        for jj in range(nch):
            a = (E_BLOCK_CH * blk + jj) % N_EACC
            if add_desc[a] is not None:
                add_desc[a].wait()
            add_desc[a] = pltpu.async_copy(
                rows[cur].at[pl.ds(jj * E_CHUNK, E_CHUNK), :],
                eacc[a].at[seg_v.at[blk * E_BLOCK_CH + jj]],
                sem_add[a], add=True)
        if blk + 1 < NBLK:
            if blk >= 1:
                for jj in range(E_BLOCK_CH):
                    sl = (E_BLOCK_CH * (blk - 1) + jj) % N_EACC
                    if add_desc[sl] is not None:
                        add_desc[sl].wait()
                        add_desc[sl] = None
            in_desc[blk + 1] = start_load(blk + 1)
    for d in add_desc:
        if d is not None:
            d.wait()

    def node_add(t):
        pltpu.sync_copy(xrows_v, nacc.at[nseg_v.at[t]], add=True)

    pltpu.sync_copy(x_hbm.at[pl.ds(wid * 128, 128), :], xrows_v)
    d_nseg0.wait()
    d_nseg1.wait()
    node_add(0)
    pltpu.sync_copy(x_hbm.at[pl.ds((wid + NW) * 128, 128), :], xrows_v)
    node_add(1)

    @pl.when(wid < N_CHUNKS_FULL - 2 * NW)
    def _third():
        q = wid + 2 * NW
        pltpu.sync_copy(batch_hbm.at[pl.ds(q * 128, 128)], nseg_v.at[2])
        pltpu.sync_copy(x_hbm.at[pl.ds(q * 128, 128), :], xrows_v)
        node_add(2)

    @pl.when(wid == NW - 1)
    def _tail():
        base = N_CHUNKS_FULL * 128
        pltpu.sync_copy(batch_hbm.at[pl.ds(base, N_TAIL)],
                        nseg_v.at[2, pl.ds(0, N_TAIL)])
        for k in range(N_TAIL // 16, 8):
            nseg_v[2, pl.ds(k * 16, 16)] = jnp.full((16,), DUMMY, jnp.int32)
        pltpu.sync_copy(x_hbm.at[pl.ds(base, N_TAIL), :],
                        xrows_v.at[pl.ds(0, N_TAIL), :])
        node_add(2)

    plsc.subcore_barrier()

    @pl.when(s == 0)
    def _out():
        for a in range(N_EACC):
            pltpu.sync_copy(eacc[a], pe_hbm.at[c, a])
        pltpu.sync_copy(nacc, pn_hbm.at[c])


def _sc_aggregate(x, edge_index, edge_attr, batch):
    mesh = plsc.VectorSubcoreMesh(core_axis_name="c", subcore_axis_name="s")
    f32 = jnp.float32
    kern = pl.kernel(
        _sc_body,
        out_type=(
            jax.ShapeDtypeStruct((NC, N_EACC, B_GRAPHS + 1, D_EDGE), f32),
            jax.ShapeDtypeStruct((NC, B_GRAPHS + 1, D_FEAT), f32),
        ),
        mesh=mesh,
        compiler_params=pltpu.CompilerParams(
            needs_layout_passes=False, use_tc_tiling_on_sc=True),
        scratch_types=[
            pltpu.VMEM((E_PER_W,), jnp.int32),
            pltpu.VMEM((N_NODES,), jnp.int32),
            pltpu.VMEM((E_NCH, E_CHUNK), jnp.int32),
            pltpu.VMEM((E_BLOCK_CH * E_CHUNK, D_EDGE), f32),
            pltpu.VMEM((E_BLOCK_CH * E_CHUNK, D_EDGE), f32),
            pltpu.VMEM((128, D_FEAT), f32),
            pltpu.VMEM((3, 128), jnp.int32),
            pltpu.VMEM((B_GRAPHS + 1, D_EDGE), f32),
            pltpu.VMEM_SHARED((B_GRAPHS + 1, D_EDGE), f32),
            pltpu.VMEM_SHARED((B_GRAPHS + 1, D_EDGE), f32),
            pltpu.VMEM_SHARED((B_GRAPHS + 1, D_EDGE), f32),
            pltpu.VMEM_SHARED((B_GRAPHS + 1, D_EDGE), f32),
            pltpu.VMEM_SHARED((B_GRAPHS + 1, D_FEAT), f32),
            pltpu.SemaphoreType.DMA,
            pltpu.SemaphoreType.DMA,
            pltpu.SemaphoreType.DMA,
            pltpu.SemaphoreType.DMA,
            pltpu.SemaphoreType.DMA,
            pltpu.SemaphoreType.DMA,
            pltpu.SemaphoreType.DMA,
            pltpu.SemaphoreType.DMA,
        ],
    )
    return kern(x, edge_index[1], edge_attr, batch)


def _mlp_body(u_ref, pe_ref, pn_ref, w1_ref, b1_ref, w2_ref, b2_ref, o_ref):
    hi = jax.lax.Precision.HIGHEST
    pe = pe_ref[...]
    agg_e = (pe[0, 0, :B_GRAPHS, :] + pe[0, 1, :B_GRAPHS, :]
             + pe[0, 2, :B_GRAPHS, :] + pe[0, 3, :B_GRAPHS, :]
             + pe[1, 0, :B_GRAPHS, :] + pe[1, 1, :B_GRAPHS, :]
             + pe[1, 2, :B_GRAPHS, :] + pe[1, 3, :B_GRAPHS, :])
    agg_n = pn_ref[0, :B_GRAPHS, :] + pn_ref[1, :B_GRAPHS, :]
    w1 = w1_ref[...]
    dn = (((1,), (0,)), ((), ()))
    z = (lax.dot_general(u_ref[...], w1[:U_DIM, :], dn, precision=hi)
         + lax.dot_general(agg_e, w1[U_DIM:U_DIM + D_EDGE, :], dn, precision=hi)
         + lax.dot_general(agg_n, w1[U_DIM + D_EDGE:, :], dn, precision=hi)
         + b1_ref[...][None, :])
    h = z * jax.nn.sigmoid(z)
    z2 = lax.dot_general(h, w2_ref[...], dn, precision=hi) + b2_ref[...][None, :]
    o_ref[...] = z2 * jax.nn.sigmoid(z2)


def _tc_mlp(u, pe, pn, W1, b1, W2, b2):
    return pl.pallas_call(
        _mlp_body,
        out_shape=jax.ShapeDtypeStruct((B_GRAPHS, K), jnp.float32),
    )(u, pe, pn, W1, b1, W2, b2)


@jax.jit
def kernel(x, edge_index, edge_attr, u, batch, W1, b1, W2, b2):
    pe, pn = _sc_aggregate(x, edge_index, edge_attr, batch)
    return _tc_mlp(u, pe, pn, W1, b1, W2, b2)


# speedup vs baseline: 1.2136x; 1.2136x over previous
import jax
import jax.numpy as jnp
from jax import lax
from jax.experimental import pallas as pl
from jax.experimental.pallas import tpu as pltpu
from jax.experimental.pallas import tpu_sc as plsc

N_NODES = 10000
N_EDGES = 320000
D_FEAT = 128
D_EDGE = 16
U_DIM = 16
B_GRAPHS = 64
K = 64

NC = 2
NS = 16
NW = NC * NS
E_PER_W = N_EDGES // NW
E_CHUNK = 128
E_ROWS_FULL = 78
E_NCH = E_ROWS_FULL + 1
E_BLOCK_CH = 16
N_CHUNKS_FULL = N_NODES // 128
N_TAIL = N_NODES - N_CHUNKS_FULL * 128
DUMMY = B_GRAPHS
N_EACC = 4

_SC_PARAMS = pltpu.CompilerParams(
    needs_layout_passes=False, use_tc_tiling_on_sc=False)


def _seg_nodes_body(x_hbm, colidx_hbm, batch_hbm, seg_hbm, pn_hbm,
                    col_v, batch_v, seg_v, xrows_v, nseg_v, zn_v, nacc,
                    sem_misc, sem_x, sem_n):
    c = lax.axis_index("c")
    s = lax.axis_index("s")
    wid = s * NC + c
    ebase = wid * E_PER_W

    d_batch = pltpu.async_copy(batch_hbm, batch_v, sem_misc)
    d_col = pltpu.async_copy(
        colidx_hbm.at[pl.ds(ebase, E_PER_W)], col_v, sem_misc)
    d_x0 = pltpu.async_copy(
        x_hbm.at[pl.ds(wid * 128, 128), :], xrows_v, sem_x)
    d_nseg0 = pltpu.async_copy(
        batch_hbm.at[pl.ds(wid * 128, 128)], nseg_v.at[0], sem_n)
    d_nseg1 = pltpu.async_copy(
        batch_hbm.at[pl.ds((wid + NW) * 128, 128)], nseg_v.at[1], sem_n)

    @pl.when(s == 0)
    def _zero():
        def zrow(r, carry):
            for k in range(D_FEAT // 16):
                zn_v[r, pl.ds(k * 16, 16)] = jnp.zeros((16,), jnp.float32)
            return carry
        lax.fori_loop(0, B_GRAPHS + 1, zrow, 0)
        pltpu.sync_copy(zn_v, nacc)

    plsc.subcore_barrier()

    d_batch.wait()
    d_col.wait()

    def seg_row(r, carry):
        for k in range(8):
            col16 = col_v[pl.ds(r * 128 + k * 16, 16)]
            seg_v[r, pl.ds(k * 16, 16)] = plsc.load_gather(batch_v, [col16])
        return carry
    lax.fori_loop(0, E_ROWS_FULL, seg_row, 0)
    col16 = col_v[pl.ds(E_ROWS_FULL * 128, 16)]
    seg_v[E_ROWS_FULL, pl.ds(0, 16)] = plsc.load_gather(batch_v, [col16])
    for k in range(1, 8):
        seg_v[E_ROWS_FULL, pl.ds(k * 16, 16)] = jnp.full((16,), DUMMY, jnp.int32)
    pltpu.sync_copy(seg_v, seg_hbm.at[wid])

    def node_add(t):
        pltpu.sync_copy(xrows_v, nacc.at[nseg_v.at[t]], add=True)

    d_x0.wait()
    d_nseg0.wait()
    d_nseg1.wait()
    node_add(0)
    pltpu.sync_copy(x_hbm.at[pl.ds((wid + NW) * 128, 128), :], xrows_v)
    node_add(1)

    @pl.when(wid < N_CHUNKS_FULL - 2 * NW)
    def _third():
        q = wid + 2 * NW
        pltpu.sync_copy(batch_hbm.at[pl.ds(q * 128, 128)], nseg_v.at[2])
        pltpu.sync_copy(x_hbm.at[pl.ds(q * 128, 128), :], xrows_v)
        node_add(2)

    @pl.when(wid == NW - 1)
    def _tail():
        base = N_CHUNKS_FULL * 128
        pltpu.sync_copy(batch_hbm.at[pl.ds(base, N_TAIL)],
                        nseg_v.at[2, pl.ds(0, N_TAIL)])
        for k in range(N_TAIL // 16, 8):
            nseg_v[2, pl.ds(k * 16, 16)] = jnp.full((16,), DUMMY, jnp.int32)
        pltpu.sync_copy(x_hbm.at[pl.ds(base, N_TAIL), :],
                        xrows_v.at[pl.ds(0, N_TAIL), :])
        node_add(2)

    plsc.subcore_barrier()

    @pl.when(s == 0)
    def _out():
        pltpu.sync_copy(nacc, pn_hbm.at[c])


def _edges_body(ea_hbm, seg_hbm, pe_hbm,
                seg_v, rows0_v, rows1_v, ze_v,
                eacc0, eacc1, eacc2, eacc3,
                sem_misc, sem_in0, sem_in1,
                sem_add0, sem_add1, sem_add2, sem_add3):
    c = lax.axis_index("c")
    s = lax.axis_index("s")
    wid = s * NC + c
    ebase = wid * E_PER_W
    rows = (rows0_v, rows1_v)
    eacc = (eacc0, eacc1, eacc2, eacc3)
    sem_in = (sem_in0, sem_in1)
    sem_add = (sem_add0, sem_add1, sem_add2, sem_add3)
    BLK = E_BLOCK_CH * E_CHUNK
    NBLK = 5
    nvalid_tail = E_PER_W - 4 * BLK

    def start_load(blk):
        buf = rows[blk % 2]
        if blk < NBLK - 1:
            return pltpu.async_copy(
                ea_hbm.at[pl.ds(ebase + blk * BLK, BLK), :], buf, sem_in[blk % 2])
        return pltpu.async_copy(
            ea_hbm.at[pl.ds(ebase + 4 * BLK, nvalid_tail), :],
            buf.at[pl.ds(0, nvalid_tail), :], sem_in[blk % 2])

    d_seg = pltpu.async_copy(seg_hbm.at[wid], seg_v, sem_misc)
    d_in0 = start_load(0)
    d_in1 = start_load(1)

    @pl.when(s == 0)
    def _zero():
        def zrow(r, carry):
            ze_v[r, pl.ds(0, 16)] = jnp.zeros((16,), jnp.float32)
            return carry
        lax.fori_loop(0, B_GRAPHS + 1, zrow, 0)
        for a in range(N_EACC):
            pltpu.sync_copy(ze_v, eacc[a])

    plsc.subcore_barrier()
    d_seg.wait()

    in_desc = [d_in0, d_in1, None, None, None]
    add_desc = [None] * N_EACC
    for blk in range(NBLK):
        cur = blk % 2
        in_desc[blk].wait()
        nch = E_BLOCK_CH if blk < 4 else E_NCH - 4 * E_BLOCK_CH
        for jj in range(nch):
            a = jj % N_EACC
            if add_desc[a] is not None:
                add_desc[a].wait()
            add_desc[a] = pltpu.async_copy(
                rows[cur].at[pl.ds(jj * E_CHUNK, E_CHUNK), :],
                eacc[a].at[seg_v.at[blk * E_BLOCK_CH + jj]],
                sem_add[a], add=True)
            if jj == N_EACC - 1 and blk + 1 < NBLK:
                in_desc[blk + 1] = start_load(blk + 1)
    for d in add_desc:
        d.wait()

    plsc.subcore_barrier()

    @pl.when(s == 0)
    def _out():
        for a in range(N_EACC):
            pltpu.sync_copy(eacc[a], pe_hbm.at[c, a])


def _sc_aggregate(x, edge_index, edge_attr, batch):
    mesh = plsc.VectorSubcoreMesh(core_axis_name="c", subcore_axis_name="s")
    f32 = jnp.float32
    i32 = jnp.int32
    kern_a = pl.kernel(
        _seg_nodes_body,
        out_type=(
            jax.ShapeDtypeStruct((NW, E_NCH, E_CHUNK), i32),
            jax.ShapeDtypeStruct((NC, B_GRAPHS + 1, D_FEAT), f32),
        ),
        mesh=mesh,
        compiler_params=_SC_PARAMS,
        scratch_types=[
            pltpu.VMEM((E_PER_W,), i32),
            pltpu.VMEM((N_NODES,), i32),
            pltpu.VMEM((E_NCH, E_CHUNK), i32),
            pltpu.VMEM((128, D_FEAT), f32),
            pltpu.VMEM((3, 128), i32),
            pltpu.VMEM((B_GRAPHS + 1, D_FEAT), f32),
            pltpu.VMEM_SHARED((B_GRAPHS + 1, D_FEAT), f32),
            pltpu.SemaphoreType.DMA,
            pltpu.SemaphoreType.DMA,
            pltpu.SemaphoreType.DMA,
        ],
    )
    seg_all, pn = kern_a(x, edge_index[1], batch)

    kern_b = pl.kernel(
        _edges_body,
        out_type=jax.ShapeDtypeStruct((NC, N_EACC, B_GRAPHS + 1, D_EDGE), f32),
        mesh=mesh,
        compiler_params=_SC_PARAMS,
        scratch_types=[
            pltpu.VMEM((E_NCH, E_CHUNK), i32),
            pltpu.VMEM((E_BLOCK_CH * E_CHUNK, D_EDGE), f32),
            pltpu.VMEM((E_BLOCK_CH * E_CHUNK, D_EDGE), f32),
            pltpu.VMEM((B_GRAPHS + 1, D_EDGE), f32),
            pltpu.VMEM_SHARED((B_GRAPHS + 1, D_EDGE), f32),
            pltpu.VMEM_SHARED((B_GRAPHS + 1, D_EDGE), f32),
            pltpu.VMEM_SHARED((B_GRAPHS + 1, D_EDGE), f32),
            pltpu.VMEM_SHARED((B_GRAPHS + 1, D_EDGE), f32),
            pltpu.SemaphoreType.DMA,
            pltpu.SemaphoreType.DMA,
            pltpu.SemaphoreType.DMA,
            pltpu.SemaphoreType.DMA,
            pltpu.SemaphoreType.DMA,
            pltpu.SemaphoreType.DMA,
            pltpu.SemaphoreType.DMA,
        ],
    )
    pe = kern_b(edge_attr, seg_all)
    return pe, pn


def _mlp_body(u_ref, pe_ref, pn_ref, w1_ref, b1_ref, w2_ref, b2_ref, o_ref):
    hi = jax.lax.Precision.HIGHEST
    pe = pe_ref[...]
    agg_e = (pe[0, 0, :B_GRAPHS, :] + pe[0, 1, :B_GRAPHS, :]
             + pe[0, 2, :B_GRAPHS, :] + pe[0, 3, :B_GRAPHS, :]
             + pe[1, 0, :B_GRAPHS, :] + pe[1, 1, :B_GRAPHS, :]
             + pe[1, 2, :B_GRAPHS, :] + pe[1, 3, :B_GRAPHS, :])
    agg_n = pn_ref[0, :B_GRAPHS, :] + pn_ref[1, :B_GRAPHS, :]
    w1 = w1_ref[...]
    dn = (((1,), (0,)), ((), ()))
    z = (lax.dot_general(u_ref[...], w1[:U_DIM, :], dn, precision=hi)
         + lax.dot_general(agg_e, w1[U_DIM:U_DIM + D_EDGE, :], dn, precision=hi)
         + lax.dot_general(agg_n, w1[U_DIM + D_EDGE:, :], dn, precision=hi)
         + b1_ref[...][None, :])
    h = z * jax.nn.sigmoid(z)
    z2 = lax.dot_general(h, w2_ref[...], dn, precision=hi) + b2_ref[...][None, :]
    o_ref[...] = z2 * jax.nn.sigmoid(z2)


def _tc_mlp(u, pe, pn, W1, b1, W2, b2):
    return pl.pallas_call(
        _mlp_body,
        out_shape=jax.ShapeDtypeStruct((B_GRAPHS, K), jnp.float32),
    )(u, pe, pn, W1, b1, W2, b2)


@jax.jit
def kernel(x, edge_index, edge_attr, u, batch, W1, b1, W2, b2):
    pe, pn = _sc_aggregate(x, edge_index, edge_attr, batch)
    return _tc_mlp(u, pe, pn, W1, b1, W2, b2)
